# initial kernel scaffold (unmeasured)
import jax
import jax.numpy as jnp
from jax import lax
from jax.experimental import pallas as pl
from jax.experimental.pallas import tpu as pltpu


def kernel(x, W, labels):
    T, D = x.shape
    _, V = W.shape
    CH = 2048

    labels2d = labels.reshape(T, 1)

    def body(x_ref, w_ref, lab_ref, out_ref, stats_ref, peer_ref,
             send_sem, recv_sem):
        my_x = lax.axis_index("x")
        my_y = lax.axis_index("y")

        barrier_sem = pltpu.get_barrier_semaphore()
        pl.semaphore_signal(
            barrier_sem, inc=1,
            device_id=(my_x, 1 - my_y), device_id_type=pl.DeviceIdType.MESH,
        )
        pl.semaphore_wait(barrier_sem, 1)

        xb = x_ref[:].astype(jnp.bfloat16)
        local_lab = lab_ref[:] - my_y * V

        def step(c, carry):
            m_run, s_run, t_run = carry
            wb = w_ref[:, pl.ds(c * CH, CH)].astype(jnp.bfloat16)
            logits = jnp.dot(xb, wb, preferred_element_type=jnp.float32)
            m_c = jnp.max(logits, axis=1, keepdims=True)
            m_new = jnp.maximum(m_run, m_c)
            s_run = s_run * jnp.exp(m_run - m_new) + jnp.sum(
                jnp.exp(logits - m_new), axis=1, keepdims=True)
            col = lax.broadcasted_iota(jnp.int32, (T, CH), 1) + c * CH
            t_run = t_run + jnp.sum(
                jnp.where(col == local_lab, logits, 0.0), axis=1, keepdims=True)
            return m_new, s_run, t_run

        neg = jnp.full((T, 1), -1e30, jnp.float32)
        zero = jnp.zeros((T, 1), jnp.float32)
        m_run, s_run, t_run = lax.fori_loop(0, V // CH, step, (neg, zero, zero))

        stats_ref[0:1, :] = m_run.T
        stats_ref[1:2, :] = s_run.T
        stats_ref[2:3, :] = t_run.T

        rdma = pltpu.make_async_remote_copy(
            src_ref=stats_ref,
            dst_ref=peer_ref,
            send_sem=send_sem,
            recv_sem=recv_sem,
            device_id=(my_x, 1 - my_y),
            device_id_type=pl.DeviceIdType.MESH,
        )
        rdma.start()
        rdma.wait()

        m0 = stats_ref[0:1, :]
        s0 = stats_ref[1:2, :]
        t0 = stats_ref[2:3, :]
        m1 = peer_ref[0:1, :]
        s1 = peer_ref[1:2, :]
        t1 = peer_ref[2:3, :]
        m = jnp.maximum(m0, m1)
        s = s0 * jnp.exp(m0 - m) + s1 * jnp.exp(m1 - m)
        out_ref[:, :] = m + jnp.log(s) - (t0 + t1)

    out = pl.pallas_call(
        body,
        out_shape=jax.ShapeDtypeStruct((1, T), jnp.float32),
        in_specs=[
            pl.BlockSpec(memory_space=pltpu.VMEM),
            pl.BlockSpec(memory_space=pltpu.VMEM),
            pl.BlockSpec(memory_space=pltpu.VMEM),
        ],
        out_specs=pl.BlockSpec(memory_space=pltpu.VMEM),
        scratch_shapes=[
            pltpu.VMEM((3, T), jnp.float32),
            pltpu.VMEM((3, T), jnp.float32),
            pltpu.SemaphoreType.DMA,
            pltpu.SemaphoreType.DMA,
        ],
        compiler_params=pltpu.CompilerParams(collective_id=0),
    )(x, W, labels2d)
    return out.reshape(T)


# baseline (device time: 18182 ns/iter reference)
import jax
import jax.numpy as jnp
from jax import lax
from jax.experimental import pallas as pl
from jax.experimental.pallas import tpu as pltpu

_PEER_OFFSETS = ((0, 1), (1, 0), (1, 1))


def kernel(x, W, labels):
    T, D = x.shape
    _, V = W.shape
    half = V // 2
    CH = 1024
    NCH = half // CH

    labels2d = labels.reshape(T, 1)

    def body(x_ref, w_hbm, lab_ref, out_ref,
             wbuf, mybuf, peerstats, wsems, send_sems, recv_sems):
        my_x = lax.axis_index("x")
        my_y = lax.axis_index("y")

        barrier_sem = pltpu.get_barrier_semaphore()
        for dx, dy in _PEER_OFFSETS:
            px = (1 - my_x) if dx else my_x
            py = (1 - my_y) if dy else my_y
            pl.semaphore_signal(
                barrier_sem, inc=1,
                device_id=(px, py), device_id_type=pl.DeviceIdType.MESH,
            )
        pl.semaphore_wait(barrier_sem, 3)

        base = my_x * half

        def w_dma(c, slot):
            return pltpu.make_async_copy(
                w_hbm.at[:, pl.ds(base + c * CH, CH)],
                wbuf.at[slot],
                wsems.at[slot],
            )

        w_dma(0, 0).start()

        xb = x_ref[:].astype(jnp.bfloat16)
        lab = lab_ref[:]
        m_run = jnp.full((T, 1), -1e30, jnp.float32)
        s_run = jnp.zeros((T, 1), jnp.float32)
        t_run = jnp.zeros((T, 1), jnp.float32)

        for c in range(NCH):
            slot = c % 2
            if c + 1 < NCH:
                w_dma(c + 1, (c + 1) % 2).start()
            w_dma(c, slot).wait()
            wb = wbuf[slot].astype(jnp.bfloat16)
            logits = jnp.dot(xb, wb, preferred_element_type=jnp.float32)
            m_c = jnp.max(logits, axis=1, keepdims=True)
            m_new = jnp.maximum(m_run, m_c)
            s_run = s_run * jnp.exp(m_run - m_new) + jnp.sum(
                jnp.exp(logits - m_new), axis=1, keepdims=True)
            glob_off = my_y * V + base + c * CH
            col = lax.broadcasted_iota(jnp.int32, (T, CH), 1) + glob_off
            t_run = t_run + jnp.sum(
                jnp.where(col == lab, logits, 0.0), axis=1, keepdims=True)
            m_run = m_new

        mybuf[0:1, :] = m_run.T
        mybuf[1:2, :] = s_run.T
        mybuf[2:3, :] = t_run.T

        rdmas = []
        for k, (dx, dy) in enumerate(_PEER_OFFSETS):
            px = (1 - my_x) if dx else my_x
            py = (1 - my_y) if dy else my_y
            r = pltpu.make_async_remote_copy(
                src_ref=mybuf,
                dst_ref=peerstats.at[k],
                send_sem=send_sems.at[k],
                recv_sem=recv_sems.at[k],
                device_id=(px, py),
                device_id_type=pl.DeviceIdType.MESH,
            )
            r.start()
            rdmas.append(r)
        for r in rdmas:
            r.wait()

        ms = [mybuf[0:1, :]] + [peerstats[k, 0:1, :] for k in range(3)]
        ss = [mybuf[1:2, :]] + [peerstats[k, 1:2, :] for k in range(3)]
        ts = [mybuf[2:3, :]] + [peerstats[k, 2:3, :] for k in range(3)]
        m = jnp.maximum(jnp.maximum(ms[0], ms[1]), jnp.maximum(ms[2], ms[3]))
        s = sum(s_i * jnp.exp(m_i - m) for s_i, m_i in zip(ss, ms))
        t = ts[0] + ts[1] + ts[2] + ts[3]
        out_ref[:, :] = m + jnp.log(s) - t

    out = pl.pallas_call(
        body,
        out_shape=jax.ShapeDtypeStruct((1, T), jnp.float32),
        in_specs=[
            pl.BlockSpec(memory_space=pltpu.VMEM),
            pl.BlockSpec(memory_space=pl.ANY),
            pl.BlockSpec(memory_space=pltpu.VMEM),
        ],
        out_specs=pl.BlockSpec(memory_space=pltpu.VMEM),
        scratch_shapes=[
            pltpu.VMEM((2, D, CH), jnp.float32),
            pltpu.VMEM((3, T), jnp.float32),
            pltpu.VMEM((3, 3, T), jnp.float32),
            pltpu.SemaphoreType.DMA((2,)),
            pltpu.SemaphoreType.DMA((3,)),
            pltpu.SemaphoreType.DMA((3,)),
        ],
        compiler_params=pltpu.CompilerParams(collective_id=0),
    )(x, W, labels2d)
    return out.reshape(T)


# device time: 16788 ns/iter; 1.0830x vs baseline; 1.0830x over previous
import os

import jax
import jax.numpy as jnp
from jax import lax
from jax.experimental import pallas as pl
from jax.experimental.pallas import tpu as pltpu

_PEER_OFFSETS = ((0, 1), (1, 0), (1, 1))

_KVAR = os.environ.get("KVAR", "")


def kernel(x, W, labels):
    T, D = x.shape
    _, V = W.shape
    half = V // 2
    CH = 1024
    NCH = half // CH

    labels2d = labels.reshape(T, 1)

    def body(x_ref, w_hbm, lab_ref, out_ref,
             wbuf, mybuf, peerstats, wsems, send_sems, recv_sems):
        my_x = lax.axis_index("x")
        my_y = lax.axis_index("y")

        barrier_sem = pltpu.get_barrier_semaphore()
        for dx, dy in _PEER_OFFSETS:
            px = (1 - my_x) if dx else my_x
            py = (1 - my_y) if dy else my_y
            pl.semaphore_signal(
                barrier_sem, inc=1,
                device_id=(px, py), device_id_type=pl.DeviceIdType.MESH,
            )
        pl.semaphore_wait(barrier_sem, 3)

        base = my_x * half

        def w_dma(c, slot):
            return pltpu.make_async_copy(
                w_hbm.at[:, pl.ds(base + c * CH, CH)],
                wbuf.at[slot],
                wsems.at[slot],
            )

        w_dma(0, 0).start()

        xb = x_ref[:].astype(jnp.bfloat16)
        lab = lab_ref[:]
        m_run = jnp.full((T, 1), -1e30, jnp.float32)
        s_run = jnp.zeros((T, 1), jnp.float32)
        t_run = jnp.zeros((T, 1), jnp.float32)

        for c in range(NCH):
            slot = c % 2
            if c + 1 < NCH:
                w_dma(c + 1, (c + 1) % 2).start()
            w_dma(c, slot).wait()
            wb = wbuf[slot].astype(jnp.bfloat16)
            logits = jnp.dot(xb, wb, preferred_element_type=jnp.float32)
            if _KVAR == "nosoftmax":
                s_run = s_run + jnp.sum(logits, axis=1, keepdims=True)
                continue
            m_c = jnp.max(logits, axis=1, keepdims=True)
            m_new = jnp.maximum(m_run, m_c)
            s_run = s_run * jnp.exp(m_run - m_new) + jnp.sum(
                jnp.exp(logits - m_new), axis=1, keepdims=True)
            glob_off = my_y * V + base + c * CH
            col = lax.broadcasted_iota(jnp.int32, (T, CH), 1) + glob_off
            t_run = t_run + jnp.sum(
                jnp.where(col == lab, logits, 0.0), axis=1, keepdims=True)
            m_run = m_new

        mybuf[0:1, :] = m_run.T
        mybuf[1:2, :] = s_run.T
        mybuf[2:3, :] = t_run.T

        if _KVAR == "nocomm":
            out_ref[:, :] = mybuf[0:1, :] + mybuf[1:2, :] + mybuf[2:3, :]
            return

        rdmas = []
        for k, (dx, dy) in enumerate(_PEER_OFFSETS):
            px = (1 - my_x) if dx else my_x
            py = (1 - my_y) if dy else my_y
            r = pltpu.make_async_remote_copy(
                src_ref=mybuf,
                dst_ref=peerstats.at[k],
                send_sem=send_sems.at[k],
                recv_sem=recv_sems.at[k],
                device_id=(px, py),
                device_id_type=pl.DeviceIdType.MESH,
            )
            r.start()
            rdmas.append(r)
        for r in rdmas:
            r.wait()

        ms = [mybuf[0:1, :]] + [peerstats[k, 0:1, :] for k in range(3)]
        ss = [mybuf[1:2, :]] + [peerstats[k, 1:2, :] for k in range(3)]
        ts = [mybuf[2:3, :]] + [peerstats[k, 2:3, :] for k in range(3)]
        m = jnp.maximum(jnp.maximum(ms[0], ms[1]), jnp.maximum(ms[2], ms[3]))
        s = sum(s_i * jnp.exp(m_i - m) for s_i, m_i in zip(ss, ms))
        t = ts[0] + ts[1] + ts[2] + ts[3]
        out_ref[:, :] = m + jnp.log(s) - t

    out = pl.pallas_call(
        body,
        out_shape=jax.ShapeDtypeStruct((1, T), jnp.float32),
        in_specs=[
            pl.BlockSpec(memory_space=pltpu.VMEM),
            pl.BlockSpec(memory_space=pl.ANY),
            pl.BlockSpec(memory_space=pltpu.VMEM),
        ],
        out_specs=pl.BlockSpec(memory_space=pltpu.VMEM),
        scratch_shapes=[
            pltpu.VMEM((2, D, CH), jnp.float32),
            pltpu.VMEM((3, T), jnp.float32),
            pltpu.VMEM((3, 3, T), jnp.float32),
            pltpu.SemaphoreType.DMA((2,)),
            pltpu.SemaphoreType.DMA((3,)),
            pltpu.SemaphoreType.DMA((3,)),
        ],
        compiler_params=pltpu.CompilerParams(collective_id=0),
    )(x, W, labels2d)
    return out.reshape(T)


# device time: 14379 ns/iter; 1.2645x vs baseline; 1.1675x over previous
import os

import jax
import jax.numpy as jnp
from jax import lax
from jax.experimental import pallas as pl
from jax.experimental.pallas import tpu as pltpu

_PEER_OFFSETS = ((0, 1), (1, 0), (1, 1))

_KVAR = os.environ.get("KVAR", "")


def kernel(x, W, labels):
    T, D = x.shape
    _, V = W.shape
    half = V // 2
    CH = 1024
    NCH = half // CH

    labels2d = labels.reshape(T, 1)

    def body(x_ref, w_hbm, lab_ref, out_ref,
             wbuf, mybuf, peerstats, wsems, send_sems, recv_sems):
        my_x = lax.axis_index("x")
        my_y = lax.axis_index("y")

        barrier_sem = pltpu.get_barrier_semaphore()
        for dx, dy in _PEER_OFFSETS:
            px = (1 - my_x) if dx else my_x
            py = (1 - my_y) if dy else my_y
            pl.semaphore_signal(
                barrier_sem, inc=1,
                device_id=(px, py), device_id_type=pl.DeviceIdType.MESH,
            )
        pl.semaphore_wait(barrier_sem, 3)

        base = my_x * half

        def w_dma(c, slot):
            return pltpu.make_async_copy(
                w_hbm.at[:, pl.ds(base + c * CH, CH)],
                wbuf.at[slot],
                wsems.at[slot],
            )

        w_dma(0, 0).start()

        xb = x_ref[:].astype(jnp.bfloat16)
        lab = lab_ref[:]
        m_run = jnp.full((T, 1), -1e30, jnp.float32)
        s_run = jnp.zeros((T, 1), jnp.float32)
        t_run = jnp.zeros((T, 1), jnp.float32)

        for c in range(NCH):
            slot = c % 2
            if c + 1 < NCH:
                w_dma(c + 1, (c + 1) % 2).start()
            w_dma(c, slot).wait()
            if _KVAR == "nomatmul":
                logits = wbuf[slot][0:T, :]
            else:
                wb = wbuf[slot].astype(jnp.bfloat16)
                logits = jnp.dot(xb, wb, preferred_element_type=jnp.float32)
            if _KVAR in ("nosoftmax", "nomatmul"):
                s_run = s_run + jnp.sum(logits, axis=1, keepdims=True)
                continue
            m_c = jnp.max(logits, axis=1, keepdims=True)
            m_new = jnp.maximum(m_run, m_c)
            s_run = s_run * jnp.exp(m_run - m_new) + jnp.sum(
                jnp.exp(logits - m_new), axis=1, keepdims=True)
            glob_off = my_y * V + base + c * CH
            col = lax.broadcasted_iota(jnp.int32, (T, CH), 1) + glob_off
            t_run = t_run + jnp.sum(
                jnp.where(col == lab, logits, 0.0), axis=1, keepdims=True)
            m_run = m_new

        mybuf[0:1, :] = m_run.T
        mybuf[1:2, :] = s_run.T
        mybuf[2:3, :] = t_run.T

        if _KVAR == "nocomm":
            out_ref[:, :] = mybuf[0:1, :] + mybuf[1:2, :] + mybuf[2:3, :]
            return

        rdmas = []
        for k, (dx, dy) in enumerate(_PEER_OFFSETS):
            px = (1 - my_x) if dx else my_x
            py = (1 - my_y) if dy else my_y
            r = pltpu.make_async_remote_copy(
                src_ref=mybuf,
                dst_ref=peerstats.at[k],
                send_sem=send_sems.at[k],
                recv_sem=recv_sems.at[k],
                device_id=(px, py),
                device_id_type=pl.DeviceIdType.MESH,
            )
            r.start()
            rdmas.append(r)
        for r in rdmas:
            r.wait()

        ms = [mybuf[0:1, :]] + [peerstats[k, 0:1, :] for k in range(3)]
        ss = [mybuf[1:2, :]] + [peerstats[k, 1:2, :] for k in range(3)]
        ts = [mybuf[2:3, :]] + [peerstats[k, 2:3, :] for k in range(3)]
        m = jnp.maximum(jnp.maximum(ms[0], ms[1]), jnp.maximum(ms[2], ms[3]))
        s = sum(s_i * jnp.exp(m_i - m) for s_i, m_i in zip(ss, ms))
        t = ts[0] + ts[1] + ts[2] + ts[3]
        out_ref[:, :] = m + jnp.log(s) - t

    out = pl.pallas_call(
        body,
        out_shape=jax.ShapeDtypeStruct((1, T), jnp.float32),
        in_specs=[
            pl.BlockSpec(memory_space=pltpu.VMEM),
            pl.BlockSpec(memory_space=pl.ANY),
            pl.BlockSpec(memory_space=pltpu.VMEM),
        ],
        out_specs=pl.BlockSpec(memory_space=pltpu.VMEM),
        scratch_shapes=[
            pltpu.VMEM((2, D, CH), jnp.float32),
            pltpu.VMEM((3, T), jnp.float32),
            pltpu.VMEM((3, 3, T), jnp.float32),
            pltpu.SemaphoreType.DMA((2,)),
            pltpu.SemaphoreType.DMA((3,)),
            pltpu.SemaphoreType.DMA((3,)),
        ],
        compiler_params=pltpu.CompilerParams(collective_id=0),
    )(x, W, labels2d)
    return out.reshape(T)
